# Initial kernel scaffold; baseline (speedup 1.0000x reference)
import jax
import jax.numpy as jnp
from jax.experimental import pallas as pl


def kernel(x, pe_index, pe_val, W_in, b_in, W1, b1, W2, b2, W_out, b_out, WO, bO):
    raise NotImplementedError("write your pallas kernel here")


# speedup vs baseline: 3.0293x; 3.0293x over previous
import functools

import jax
import jax.numpy as jnp
from jax import lax
from jax.experimental import pallas as pl
from jax.experimental.pallas import tpu as pltpu
from jax.experimental.pallas import tpu_sc as plsc

_N, _E, _D, _P = 10000, 320000, 128, 16
_NC, _NS, _L = 2, 16, 16
_NW = _NC * _NS
_EW = _E // _NW
_C = 80
_NCHUNK = _EW // _C
_NPAD = 10240
_RS = _NPAD // _NS


def _erf(v):
    a = jnp.abs(v)
    t = 1.0 / (1.0 + 0.3275911 * a)
    poly = ((((1.061405429 * t - 1.453152027) * t + 1.421413741) * t
             - 0.284496736) * t + 0.254829592) * t
    y = 1.0 - poly * jnp.exp(-a * a)
    return jnp.sign(v) * y


def _gelu(t):
    return 0.5 * t * (1.0 + _erf(t * 0.7071067811865476))


_BE = 1280


def _mlp_body(pe_ref, Win_ref, bin_ref, W1_ref, b1_ref, W2_ref, b2_ref,
              Wout_ref, bout_ref, out_ref):
    pe = pe_ref[...]
    h0 = jnp.dot(pe, Win_ref[...], preferred_element_type=jnp.float32) + bin_ref[...]
    h = _gelu(h0)
    h = jnp.dot(h, W1_ref[...], preferred_element_type=jnp.float32) + b1_ref[...]
    h = _gelu(h)
    h = jnp.dot(h, W2_ref[...], preferred_element_type=jnp.float32) + b2_ref[...]
    hx = h0 + h
    out_ref[...] = (jnp.dot(hx, Wout_ref[...], preferred_element_type=jnp.float32)
                    + bout_ref[...])


def _edge_mlp(pe_val, W_in, b_in, W1, b1, W2, b2, W_out, b_out):
    hid = W_in.shape[1]
    return pl.pallas_call(
        _mlp_body,
        grid=(_E // _BE,),
        in_specs=[
            pl.BlockSpec((_BE, _P), lambda i: (i, 0)),
            pl.BlockSpec((_P, hid), lambda i: (0, 0)),
            pl.BlockSpec((1, hid), lambda i: (0, 0)),
            pl.BlockSpec((hid, hid), lambda i: (0, 0)),
            pl.BlockSpec((1, hid), lambda i: (0, 0)),
            pl.BlockSpec((hid, hid), lambda i: (0, 0)),
            pl.BlockSpec((1, hid), lambda i: (0, 0)),
            pl.BlockSpec((hid, _D), lambda i: (0, 0)),
            pl.BlockSpec((1, _D), lambda i: (0, 0)),
        ],
        out_specs=pl.BlockSpec((_BE, _D), lambda i: (i, 0)),
        out_shape=jax.ShapeDtypeStruct((_E, _D), jnp.float32),
    )(pe_val, W_in, b_in.reshape(1, hid), W1, b1.reshape(1, hid),
      W2, b2.reshape(1, hid), W_out, b_out.reshape(1, _D))


def _sc_body(x_hbm, score_hbm, src_hbm, dst_hbm, z2_hbm, z1_hbm,
             sums_out, cnt_out,
             idx_src, idx_dst, rows, scv, ones, sums_sp, cnt_sp, sem):
    c = lax.axis_index("c")
    s = lax.axis_index("s")
    wid = c * _NS + s
    ebase = wid * _EW
    rbase = s * _RS

    for k in range(_C // _L):
        ones[pl.ds(k * _L, _L)] = jnp.full((_L,), 1.0, jnp.float32)

    pltpu.sync_copy(z2_hbm, sums_sp.at[pl.ds(rbase, _RS), :])
    pltpu.sync_copy(z1_hbm, cnt_sp.at[pl.ds(rbase, _RS)])
    plsc.subcore_barrier()

    def chunk(i, carry):
        base = ebase + i * _C
        pltpu.sync_copy(src_hbm.at[pl.ds(base, _C)], idx_src)
        pltpu.sync_copy(dst_hbm.at[pl.ds(base, _C)], idx_dst)
        cp = pltpu.async_copy(x_hbm.at[idx_src], rows, sem)
        pltpu.sync_copy(score_hbm.at[pl.ds(base, _C)], scv)
        cp.wait()

        def mul(e, cc):
            for k in range(_D // _L):
                sl = pl.ds(k * _L, _L)
                rows[e, sl] = rows[e, sl] * scv[e, sl]
            return cc

        lax.fori_loop(0, _C, mul, 0)
        pltpu.sync_copy(rows, sums_sp.at[idx_dst], add=True)
        pltpu.sync_copy(ones, cnt_sp.at[idx_dst], add=True)
        return carry

    lax.fori_loop(0, _NCHUNK, chunk, 0)
    plsc.subcore_barrier()

    pltpu.sync_copy(sums_sp.at[pl.ds(rbase, _RS), :],
                    sums_out.at[c, pl.ds(rbase, _RS), :])
    pltpu.sync_copy(cnt_sp.at[pl.ds(rbase, _RS)],
                    cnt_out.at[c, pl.ds(rbase, _RS)])


def _sc_gather_scatter(x, score, src, dst, z2, z1):
    mesh = plsc.VectorSubcoreMesh(core_axis_name="c", subcore_axis_name="s")
    f = pl.kernel(
        _sc_body,
        out_type=[
            jax.ShapeDtypeStruct((_NC, _NPAD, _D), jnp.float32),
            jax.ShapeDtypeStruct((_NC, _NPAD), jnp.float32),
        ],
        mesh=mesh,
        scratch_types=[
            pltpu.VMEM((_C,), jnp.int32),
            pltpu.VMEM((_C,), jnp.int32),
            pltpu.VMEM((_C, _D), jnp.float32),
            pltpu.VMEM((_C, _D), jnp.float32),
            pltpu.VMEM((_C,), jnp.float32),
            pltpu.VMEM_SHARED((_NPAD, _D), jnp.float32),
            pltpu.VMEM_SHARED((_NPAD,), jnp.float32),
            pltpu.SemaphoreType.DMA,
        ],
    )
    return f(x, score, src, dst, z2, z1)


_BN = 1000


def _out_body(sums_ref, cnt_ref, WO_ref, bO_ref, out_ref):
    s = sums_ref[0] + sums_ref[1]
    cnt = cnt_ref[:, 0:1] + cnt_ref[:, 1:2]
    wv = s / jnp.maximum(cnt, 1.0)
    out_ref[...] = (jnp.dot(wv, WO_ref[...], preferred_element_type=jnp.float32)
                    + bO_ref[...])


def _mean_project(sums, cnt_t, WO, bO):
    return pl.pallas_call(
        _out_body,
        grid=(_N // _BN,),
        in_specs=[
            pl.BlockSpec((_NC, _BN, _D), lambda i: (0, i, 0)),
            pl.BlockSpec((_BN, _NC), lambda i: (i, 0)),
            pl.BlockSpec((_D, _D), lambda i: (0, 0)),
            pl.BlockSpec((1, _D), lambda i: (0, 0)),
        ],
        out_specs=pl.BlockSpec((_BN, _D), lambda i: (i, 0)),
        out_shape=jax.ShapeDtypeStruct((_N, _D), jnp.float32),
    )(sums, cnt_t, WO, bO.reshape(1, _D))


def kernel(x, pe_index, pe_val, W_in, b_in, W1, b1, W2, b2, W_out, b_out, WO, bO):
    score = _edge_mlp(pe_val, W_in, b_in, W1, b1, W2, b2, W_out, b_out)
    src = pe_index[0]
    dst = pe_index[1]
    z2 = jnp.zeros((_RS, _D), jnp.float32)
    z1 = jnp.zeros((_RS,), jnp.float32)
    sums, cnt = _sc_gather_scatter(x, score, src, dst, z2, z1)
    cnt_t = jnp.transpose(cnt[:, :_N])
    return _mean_project(sums, cnt_t, WO, bO)


# speedup vs baseline: 4.5670x; 1.5076x over previous
import jax
import jax.numpy as jnp
from jax import lax
from jax.experimental import pallas as pl
from jax.experimental.pallas import tpu as pltpu
from jax.experimental.pallas import tpu_sc as plsc

_N, _E, _D, _P = 10000, 320000, 128, 16
_NC, _NS, _L = 2, 16, 16
_NW = _NC * _NS
_EW = _E // _NW
_C = 40
_NCHUNK = _EW // _C
_NPAD = 10240
_RS = _NPAD // _NS


def _gelu(t):
    t2 = t * t
    u = t * (0.7978845608028654 + 0.03567740813636141 * t2)
    return 0.5 * t * (1.0 + jnp.tanh(u))


_BE = 1280


def _mlp_body(pe_ref, Win_ref, bin_ref, W1_ref, b1_ref, W2_ref, b2_ref,
              Wout_ref, bout_ref, out_ref):
    pe = pe_ref[...]
    h0 = jnp.dot(pe, Win_ref[...], preferred_element_type=jnp.float32) + bin_ref[...]
    h = _gelu(h0)
    h = jnp.dot(h, W1_ref[...], preferred_element_type=jnp.float32) + b1_ref[...]
    h = _gelu(h)
    h = jnp.dot(h, W2_ref[...], preferred_element_type=jnp.float32) + b2_ref[...]
    hx = h0 + h
    out_ref[...] = (jnp.dot(hx, Wout_ref[...], preferred_element_type=jnp.float32)
                    + bout_ref[...])


def _edge_mlp(pe_val, W_in, b_in, W1, b1, W2, b2, W_out, b_out):
    hid = W_in.shape[1]
    return pl.pallas_call(
        _mlp_body,
        grid=(_E // _BE,),
        in_specs=[
            pl.BlockSpec((_BE, _P), lambda i: (i, 0)),
            pl.BlockSpec((_P, hid), lambda i: (0, 0)),
            pl.BlockSpec((1, hid), lambda i: (0, 0)),
            pl.BlockSpec((hid, hid), lambda i: (0, 0)),
            pl.BlockSpec((1, hid), lambda i: (0, 0)),
            pl.BlockSpec((hid, hid), lambda i: (0, 0)),
            pl.BlockSpec((1, hid), lambda i: (0, 0)),
            pl.BlockSpec((hid, _D), lambda i: (0, 0)),
            pl.BlockSpec((1, _D), lambda i: (0, 0)),
        ],
        out_specs=pl.BlockSpec((_BE, _D), lambda i: (i, 0)),
        out_shape=jax.ShapeDtypeStruct((_E, _D), jnp.float32),
    )(pe_val, W_in, b_in.reshape(1, hid), W1, b1.reshape(1, hid),
      W2, b2.reshape(1, hid), W_out, b_out.reshape(1, _D))


def _sc_body(x_hbm, score_hbm, src_hbm, dst_hbm, z2_hbm, z1_hbm,
             sums_out, cnt_out,
             isrc, idst, rows0, rows1, scv0, scv1, ones,
             sums_sp, cnt_sp,
             sem_g0, sem_g1, sem_i):
    c = lax.axis_index("c")
    s = lax.axis_index("s")
    wid = c * _NS + s
    ebase = wid * _EW
    rbase = s * _RS

    for k in range(48 // _L):
        ones[pl.ds(k * _L, _L)] = jnp.full((_L,), 1.0, jnp.float32)
    one_src = ones.at[pl.ds(0, _C)]

    pltpu.sync_copy(z2_hbm, sums_sp.at[pl.ds(rbase, _RS), :])
    pltpu.sync_copy(z1_hbm, cnt_sp.at[pl.ds(rbase, _RS)])
    plsc.subcore_barrier()

    def fetch_idx(i, sem):
        r = lax.rem(i, 4)
        pltpu.async_copy(src_hbm.at[wid, i], isrc.at[r], sem)
        pltpu.async_copy(dst_hbm.at[wid, i], idst.at[r], sem)

    def wait_idx(i, sem):
        r = lax.rem(i, 4)
        pltpu.make_async_copy(src_hbm.at[wid, i], isrc.at[r], sem).wait()
        pltpu.make_async_copy(dst_hbm.at[wid, i], idst.at[r], sem).wait()

    def fetch(i, rows, scv, sem):
        r = lax.rem(i, 4)
        pltpu.async_copy(x_hbm.at[isrc.at[r]], rows, sem)
        pltpu.async_copy(score_hbm.at[pl.ds(ebase + i * _C, _C)], scv, sem)

    def wait_fetch(i, rows, scv, sem):
        r = lax.rem(i, 4)
        pltpu.make_async_copy(x_hbm.at[isrc.at[r]], rows, sem).wait()
        pltpu.make_async_copy(score_hbm.at[pl.ds(ebase + i * _C, _C)], scv,
                              sem).wait()

    def multiply(rows, scv):
        def mul(e, cc):
            for k in range(_D // _L):
                sl = pl.ds(k * _L, _L)
                rows[e, sl] = rows[e, sl] * scv[e, sl]
            return cc
        lax.fori_loop(0, _C, mul, 0)

    def step(i, j, rows, scv, sem_g):
        wait_fetch(i, rows, scv, sem_g)
        multiply(rows, scv)
        r = lax.rem(i, 4)
        pltpu.sync_copy(rows, sums_sp.at[idst.at[r]], add=True)
        pltpu.sync_copy(one_src, cnt_sp.at[idst.at[r]], add=True)

        @pl.when(j <= _NCHUNK // 2 - 3)
        def _():
            fetch_idx(i + 4, sem_i)

        @pl.when(j <= _NCHUNK // 2 - 2)
        def _():
            wait_idx(i + 2, sem_i)
            fetch(i + 2, rows, scv, sem_g)

    for i in range(4):
        fetch_idx(i, sem_i)
    wait_idx(0, sem_i)
    wait_idx(1, sem_i)
    fetch(0, rows0, scv0, sem_g0)
    fetch(1, rows1, scv1, sem_g1)

    def body(j, carry):
        step(2 * j, j, rows0, scv0, sem_g0)
        step(2 * j + 1, j, rows1, scv1, sem_g1)
        return carry

    lax.fori_loop(0, _NCHUNK // 2, body, 0)
    plsc.subcore_barrier()

    pltpu.sync_copy(sums_sp.at[pl.ds(rbase, _RS), :],
                    sums_out.at[c, pl.ds(rbase, _RS), :])
    pltpu.sync_copy(cnt_sp.at[pl.ds(rbase, _RS)],
                    cnt_out.at[c, pl.ds(rbase, _RS)])


def _sc_gather_scatter(x, score, src3, dst3, z2, z1):
    mesh = plsc.VectorSubcoreMesh(core_axis_name="c", subcore_axis_name="s")
    f = pl.kernel(
        _sc_body,
        out_type=[
            jax.ShapeDtypeStruct((_NC, _NPAD, _D), jnp.float32),
            jax.ShapeDtypeStruct((_NC, _NPAD), jnp.float32),
        ],
        mesh=mesh,
        scratch_types=[
            pltpu.VMEM((4, _C), jnp.int32),
            pltpu.VMEM((4, _C), jnp.int32),
            pltpu.VMEM((_C, _D), jnp.float32),
            pltpu.VMEM((_C, _D), jnp.float32),
            pltpu.VMEM((_C, _D), jnp.float32),
            pltpu.VMEM((_C, _D), jnp.float32),
            pltpu.VMEM((48,), jnp.float32),
            pltpu.VMEM_SHARED((_NPAD, _D), jnp.float32),
            pltpu.VMEM_SHARED((_NPAD,), jnp.float32),
            pltpu.SemaphoreType.DMA,
            pltpu.SemaphoreType.DMA,
            pltpu.SemaphoreType.DMA,
        ],
    )
    return f(x, score, src3, dst3, z2, z1)


_BN = 1000


def _out_body(sums_ref, cnt_ref, WO_ref, bO_ref, out_ref):
    s = sums_ref[0] + sums_ref[1]
    cnt = cnt_ref[:, 0:1] + cnt_ref[:, 1:2]
    wv = s / jnp.maximum(cnt, 1.0)
    out_ref[...] = (jnp.dot(wv, WO_ref[...], preferred_element_type=jnp.float32)
                    + bO_ref[...])


def _mean_project(sums, cnt_t, WO, bO):
    return pl.pallas_call(
        _out_body,
        grid=(_N // _BN,),
        in_specs=[
            pl.BlockSpec((_NC, _BN, _D), lambda i: (0, i, 0)),
            pl.BlockSpec((_BN, _NC), lambda i: (i, 0)),
            pl.BlockSpec((_D, _D), lambda i: (0, 0)),
            pl.BlockSpec((1, _D), lambda i: (0, 0)),
        ],
        out_specs=pl.BlockSpec((_BN, _D), lambda i: (i, 0)),
        out_shape=jax.ShapeDtypeStruct((_N, _D), jnp.float32),
    )(sums, cnt_t, WO, bO.reshape(1, _D))


def kernel(x, pe_index, pe_val, W_in, b_in, W1, b1, W2, b2, W_out, b_out, WO, bO):
    score = _edge_mlp(pe_val, W_in, b_in, W1, b1, W2, b2, W_out, b_out)
    src3 = pe_index[0].reshape(_NW, _NCHUNK, _C)
    dst3 = pe_index[1].reshape(_NW, _NCHUNK, _C)
    z2 = jnp.zeros((_RS, _D), jnp.float32)
    z1 = jnp.zeros((_RS,), jnp.float32)
    sums, cnt = _sc_gather_scatter(x, score, src3, dst3, z2, z1)
    cnt_t = jnp.transpose(cnt[:, :_N])
    return _mean_project(sums, cnt_t, WO, bO)


# speedup vs baseline: 4.8092x; 1.0530x over previous
import jax
import jax.numpy as jnp
from jax import lax
from jax.experimental import pallas as pl
from jax.experimental.pallas import tpu as pltpu
from jax.experimental.pallas import tpu_sc as plsc

_N, _E, _D, _P = 10000, 320000, 128, 16
_NC, _NS, _L = 2, 16, 16
_NW = _NC * _NS
_EW = _E // _NW
_C = 40
_NCHUNK = _EW // _C
_NPAD = 10240
_RS = _NPAD // _NS


def _gelu(t):
    t2 = t * t
    u = t * (0.7978845608028654 + 0.03567740813636141 * t2)
    return 0.5 * t * (1.0 + jnp.tanh(u))


_BE = 1280


def _mlp_body(pe_ref, Win_ref, bin_ref, W1_ref, b1_ref, W2_ref, b2_ref,
              Wout_ref, bout_ref, out_ref):
    pe = pe_ref[...]
    h0 = jnp.dot(pe, Win_ref[...], preferred_element_type=jnp.float32) + bin_ref[...]
    h = _gelu(h0)
    h = jnp.dot(h, W1_ref[...], preferred_element_type=jnp.float32) + b1_ref[...]
    h = _gelu(h)
    h = jnp.dot(h, W2_ref[...], preferred_element_type=jnp.float32) + b2_ref[...]
    hx = h0 + h
    out_ref[...] = (jnp.dot(hx, Wout_ref[...], preferred_element_type=jnp.float32)
                    + bout_ref[...])


def _edge_mlp(pe_val, W_in, b_in, W1, b1, W2, b2, W_out, b_out):
    hid = W_in.shape[1]
    return pl.pallas_call(
        _mlp_body,
        grid=(_E // _BE,),
        in_specs=[
            pl.BlockSpec((_BE, _P), lambda i: (i, 0)),
            pl.BlockSpec((_P, hid), lambda i: (0, 0)),
            pl.BlockSpec((1, hid), lambda i: (0, 0)),
            pl.BlockSpec((hid, hid), lambda i: (0, 0)),
            pl.BlockSpec((1, hid), lambda i: (0, 0)),
            pl.BlockSpec((hid, hid), lambda i: (0, 0)),
            pl.BlockSpec((1, hid), lambda i: (0, 0)),
            pl.BlockSpec((hid, _D), lambda i: (0, 0)),
            pl.BlockSpec((1, _D), lambda i: (0, 0)),
        ],
        out_specs=pl.BlockSpec((_BE, _D), lambda i: (i, 0)),
        out_shape=jax.ShapeDtypeStruct((_E, _D), jnp.float32),
    )(pe_val, W_in, b_in.reshape(1, hid), W1, b1.reshape(1, hid),
      W2, b2.reshape(1, hid), W_out, b_out.reshape(1, _D))


def _sc_body(x_hbm, score_hbm, src_hbm, dst_hbm, z2_hbm, z1_hbm,
             sums_out, cnt_out,
             isrc, idst, rows0, rows1, scv0, scv1, msg0, msg1, ones,
             sums_sp, cnt_sp,
             sem_g0, sem_g1, sem_w0, sem_w1, sem_i):
    c = lax.axis_index("c")
    s = lax.axis_index("s")
    wid = c * _NS + s
    ebase = wid * _EW
    rbase = s * _RS

    for k in range(48 // _L):
        ones[pl.ds(k * _L, _L)] = jnp.full((_L,), 1.0, jnp.float32)
    one_src = ones.at[pl.ds(0, _C)]

    pltpu.sync_copy(z2_hbm, sums_sp.at[pl.ds(rbase, _RS), :])
    pltpu.sync_copy(z1_hbm, cnt_sp.at[pl.ds(rbase, _RS)])
    plsc.subcore_barrier()

    def fetch_idx(i, sem):
        r = lax.rem(i, 8)
        pltpu.async_copy(src_hbm.at[wid, i], isrc.at[r], sem)
        pltpu.async_copy(dst_hbm.at[wid, i], idst.at[r], sem)

    def wait_idx(i, sem):
        r = lax.rem(i, 8)
        pltpu.make_async_copy(src_hbm.at[wid, i], isrc.at[r], sem).wait()
        pltpu.make_async_copy(dst_hbm.at[wid, i], idst.at[r], sem).wait()

    def fetch(i, rows, scv, sem):
        r = lax.rem(i, 8)
        pltpu.async_copy(x_hbm.at[isrc.at[r]], rows, sem)
        pltpu.async_copy(score_hbm.at[pl.ds(ebase + i * _C, _C)], scv, sem)

    def wait_fetch(i, rows, scv, sem):
        r = lax.rem(i, 8)
        pltpu.make_async_copy(x_hbm.at[isrc.at[r]], rows, sem).wait()
        pltpu.make_async_copy(score_hbm.at[pl.ds(ebase + i * _C, _C)], scv,
                              sem).wait()

    def multiply(rows, scv, msg):
        def mul(e, cc):
            for k in range(_D // _L):
                sl = pl.ds(k * _L, _L)
                msg[e, sl] = rows[e, sl] * scv[e, sl]
            return cc
        lax.fori_loop(0, _C, mul, 0)

    def wait_scatter(i, msg, sem):
        r = lax.rem(i, 8)
        pltpu.make_async_copy(msg, sums_sp.at[idst.at[r]], sem).wait()
        pltpu.make_async_copy(one_src, cnt_sp.at[idst.at[r]], sem).wait()

    def step(i, j, rows, scv, msg, sem_g, sem_w):
        wait_fetch(i, rows, scv, sem_g)

        @pl.when(j >= 1)
        def _():
            wait_scatter(i - 2, msg, sem_w)

        multiply(rows, scv, msg)
        r = lax.rem(i, 8)
        pltpu.async_copy(msg, sums_sp.at[idst.at[r]], sem_w, add=True)
        pltpu.async_copy(one_src, cnt_sp.at[idst.at[r]], sem_w, add=True)

        @pl.when(j <= _NCHUNK // 2 - 3)
        def _():
            fetch_idx(i + 4, sem_i)

        @pl.when(j <= _NCHUNK // 2 - 2)
        def _():
            wait_idx(i + 2, sem_i)
            fetch(i + 2, rows, scv, sem_g)

    for i in range(4):
        fetch_idx(i, sem_i)
    wait_idx(0, sem_i)
    wait_idx(1, sem_i)
    fetch(0, rows0, scv0, sem_g0)
    fetch(1, rows1, scv1, sem_g1)

    def body(j, carry):
        step(2 * j, j, rows0, scv0, msg0, sem_g0, sem_w0)
        step(2 * j + 1, j, rows1, scv1, msg1, sem_g1, sem_w1)
        return carry

    lax.fori_loop(0, _NCHUNK // 2, body, 0)
    wait_scatter(_NCHUNK - 2, msg0, sem_w0)
    wait_scatter(_NCHUNK - 1, msg1, sem_w1)
    plsc.subcore_barrier()

    pltpu.sync_copy(sums_sp.at[pl.ds(rbase, _RS), :],
                    sums_out.at[c, pl.ds(rbase, _RS), :])
    pltpu.sync_copy(cnt_sp.at[pl.ds(rbase, _RS)],
                    cnt_out.at[c, pl.ds(rbase, _RS)])


def _sc_gather_scatter(x, score, src3, dst3, z2, z1):
    mesh = plsc.VectorSubcoreMesh(core_axis_name="c", subcore_axis_name="s")
    f = pl.kernel(
        _sc_body,
        out_type=[
            jax.ShapeDtypeStruct((_NC, _NPAD, _D), jnp.float32),
            jax.ShapeDtypeStruct((_NC, _NPAD), jnp.float32),
        ],
        mesh=mesh,
        scratch_types=[
            pltpu.VMEM((8, _C), jnp.int32),
            pltpu.VMEM((8, _C), jnp.int32),
            pltpu.VMEM((_C, _D), jnp.float32),
            pltpu.VMEM((_C, _D), jnp.float32),
            pltpu.VMEM((_C, _D), jnp.float32),
            pltpu.VMEM((_C, _D), jnp.float32),
            pltpu.VMEM((_C, _D), jnp.float32),
            pltpu.VMEM((_C, _D), jnp.float32),
            pltpu.VMEM((48,), jnp.float32),
            pltpu.VMEM_SHARED((_NPAD, _D), jnp.float32),
            pltpu.VMEM_SHARED((_NPAD,), jnp.float32),
            pltpu.SemaphoreType.DMA,
            pltpu.SemaphoreType.DMA,
            pltpu.SemaphoreType.DMA,
            pltpu.SemaphoreType.DMA,
            pltpu.SemaphoreType.DMA,
        ],
    )
    return f(x, score, src3, dst3, z2, z1)


_BN = 1000


def _out_body(sums_ref, cnt_ref, WO_ref, bO_ref, out_ref):
    s = sums_ref[0] + sums_ref[1]
    cnt = cnt_ref[:, 0:1] + cnt_ref[:, 1:2]
    wv = s / jnp.maximum(cnt, 1.0)
    out_ref[...] = (jnp.dot(wv, WO_ref[...], preferred_element_type=jnp.float32)
                    + bO_ref[...])


def _mean_project(sums, cnt_t, WO, bO):
    return pl.pallas_call(
        _out_body,
        grid=(_N // _BN,),
        in_specs=[
            pl.BlockSpec((_NC, _BN, _D), lambda i: (0, i, 0)),
            pl.BlockSpec((_BN, _NC), lambda i: (i, 0)),
            pl.BlockSpec((_D, _D), lambda i: (0, 0)),
            pl.BlockSpec((1, _D), lambda i: (0, 0)),
        ],
        out_specs=pl.BlockSpec((_BN, _D), lambda i: (i, 0)),
        out_shape=jax.ShapeDtypeStruct((_N, _D), jnp.float32),
    )(sums, cnt_t, WO, bO.reshape(1, _D))


def kernel(x, pe_index, pe_val, W_in, b_in, W1, b1, W2, b2, W_out, b_out, WO, bO):
    score = _edge_mlp(pe_val, W_in, b_in, W1, b1, W2, b2, W_out, b_out)
    src3 = pe_index[0].reshape(_NW, _NCHUNK, _C)
    dst3 = pe_index[1].reshape(_NW, _NCHUNK, _C)
    z2 = jnp.zeros((_RS, _D), jnp.float32)
    z1 = jnp.zeros((_RS,), jnp.float32)
    sums, cnt = _sc_gather_scatter(x, score, src3, dst3, z2, z1)
    cnt_t = jnp.transpose(cnt[:, :_N])
    return _mean_project(sums, cnt_t, WO, bO)


# speedup vs baseline: 5.3564x; 1.1138x over previous
import jax
import jax.numpy as jnp
from jax import lax
from jax.experimental import pallas as pl
from jax.experimental.pallas import tpu as pltpu
from jax.experimental.pallas import tpu_sc as plsc

_N, _E, _D, _P = 10000, 320000, 128, 16
_NC, _NS, _L = 2, 16, 16
_NW = _NC * _NS
_C = 40
_NPAD = 10240
_RS = _NPAD // _NS
_S = 2


def _gelu(t):
    t2 = t * t
    u = t * (0.7978845608028654 + 0.03567740813636141 * t2)
    return 0.5 * t * (1.0 + jnp.tanh(u))


_BE = 1280


def _mlp_body(pe_ref, Win_ref, bin_ref, W1_ref, b1_ref, W2_ref, b2_ref,
              Wout_ref, bout_ref, out_ref):
    pe = pe_ref[...]
    h0p = jnp.dot(pe, Win_ref[...], preferred_element_type=jnp.float32)
    h0 = h0p.reshape(_BE, _D) + bin_ref[...]
    h = _gelu(h0)
    h = jnp.dot(h, W1_ref[...], preferred_element_type=jnp.float32) + b1_ref[...]
    h = _gelu(h)
    h = jnp.dot(h, W2_ref[...], preferred_element_type=jnp.float32) + b2_ref[...]
    hx = h0 + h
    out_ref[...] = (jnp.dot(hx, Wout_ref[...], preferred_element_type=jnp.float32)
                    + bout_ref[...])


def _edge_mlp(pe8, W_blk, b_in, W1, b1, W2, b2, W_out, b_out, nblk, blk_off):
    hid = W1.shape[0]
    return pl.pallas_call(
        _mlp_body,
        grid=(nblk,),
        in_specs=[
            pl.BlockSpec((_BE // 8, 8 * _P), lambda i, o=blk_off: (i + o, 0)),
            pl.BlockSpec((8 * _P, 8 * hid), lambda i: (0, 0)),
            pl.BlockSpec((1, hid), lambda i: (0, 0)),
            pl.BlockSpec((hid, hid), lambda i: (0, 0)),
            pl.BlockSpec((1, hid), lambda i: (0, 0)),
            pl.BlockSpec((hid, hid), lambda i: (0, 0)),
            pl.BlockSpec((1, hid), lambda i: (0, 0)),
            pl.BlockSpec((hid, _D), lambda i: (0, 0)),
            pl.BlockSpec((1, _D), lambda i: (0, 0)),
        ],
        out_specs=pl.BlockSpec((_BE, _D), lambda i: (i, 0)),
        out_shape=jax.ShapeDtypeStruct((nblk * _BE, _D), jnp.float32),
    )(pe8, W_blk, b_in.reshape(1, hid), W1, b1.reshape(1, hid),
      W2, b2.reshape(1, hid), W_out, b_out.reshape(1, _D))


def _make_sc_body(nchunk):
    def _sc_body(x_hbm, score_hbm, src_hbm, dst_hbm, z2_hbm, z1_hbm,
                 sums_out, cnt_out,
                 isrc, idst, rows0, rows1, scv0, scv1, msg0, msg1, ones,
                 sums_sp, cnt_sp,
                 sem_g0, sem_g1, sem_w0, sem_w1, sem_i):
        c = lax.axis_index("c")
        s = lax.axis_index("s")
        wid = c * _NS + s
        ebase = wid * (nchunk * _C)
        rbase = s * _RS

        for k in range(48 // _L):
            ones[pl.ds(k * _L, _L)] = jnp.full((_L,), 1.0, jnp.float32)
        one_src = ones.at[pl.ds(0, _C)]

        pltpu.sync_copy(z2_hbm, sums_sp.at[pl.ds(rbase, _RS), :])
        pltpu.sync_copy(z1_hbm, cnt_sp.at[pl.ds(rbase, _RS)])
        plsc.subcore_barrier()

        def fetch_idx(i, sem):
            r = lax.rem(i, 8)
            pltpu.async_copy(src_hbm.at[wid, i], isrc.at[r], sem)
            pltpu.async_copy(dst_hbm.at[wid, i], idst.at[r], sem)

        def wait_idx(i, sem):
            r = lax.rem(i, 8)
            pltpu.make_async_copy(src_hbm.at[wid, i], isrc.at[r], sem).wait()
            pltpu.make_async_copy(dst_hbm.at[wid, i], idst.at[r], sem).wait()

        def fetch(i, rows, scv, sem):
            r = lax.rem(i, 8)
            pltpu.async_copy(x_hbm.at[isrc.at[r]], rows, sem)
            pltpu.async_copy(score_hbm.at[pl.ds(ebase + i * _C, _C)], scv, sem)

        def wait_fetch(i, rows, scv, sem):
            r = lax.rem(i, 8)
            pltpu.make_async_copy(x_hbm.at[isrc.at[r]], rows, sem).wait()
            pltpu.make_async_copy(score_hbm.at[pl.ds(ebase + i * _C, _C)],
                                  scv, sem).wait()

        def multiply(rows, scv, msg):
            def mul(e8, cc):
                for u in range(8):
                    e = e8 * 8 + u
                    for k in range(_D // _L):
                        sl = pl.ds(k * _L, _L)
                        msg[e, sl] = rows[e, sl] * scv[e, sl]
                return cc
            lax.fori_loop(0, _C // 8, mul, 0)

        def wait_scatter(i, msg, sem):
            r = lax.rem(i, 8)
            pltpu.make_async_copy(msg, sums_sp.at[idst.at[r]], sem).wait()
            pltpu.make_async_copy(one_src, cnt_sp.at[idst.at[r]], sem).wait()

        def step(i, rows, scv, msg, sem_g, sem_w):
            wait_fetch(i, rows, scv, sem_g)

            @pl.when(i >= 2)
            def _():
                wait_scatter(i - 2, msg, sem_w)

            multiply(rows, scv, msg)
            r = lax.rem(i, 8)
            pltpu.async_copy(msg, sums_sp.at[idst.at[r]], sem_w, add=True)
            pltpu.async_copy(one_src, cnt_sp.at[idst.at[r]], sem_w, add=True)

            @pl.when(i + 4 < nchunk)
            def _():
                fetch_idx(i + 4, sem_i)

            @pl.when(i + 2 < nchunk)
            def _():
                wait_idx(i + 2, sem_i)
                fetch(i + 2, rows, scv, sem_g)

        for i in range(4):
            fetch_idx(i, sem_i)
        wait_idx(0, sem_i)
        wait_idx(1, sem_i)
        fetch(0, rows0, scv0, sem_g0)
        fetch(1, rows1, scv1, sem_g1)

        def body(j, carry):
            step(2 * j, rows0, scv0, msg0, sem_g0, sem_w0)
            step(2 * j + 1, rows1, scv1, msg1, sem_g1, sem_w1)
            return carry

        lax.fori_loop(0, nchunk // 2, body, 0)
        if nchunk % 2 == 1:
            step(jnp.int32(nchunk - 1), rows0, scv0, msg0, sem_g0, sem_w0)

        for t in (nchunk - 2, nchunk - 1):
            if t % 2 == 0:
                wait_scatter(jnp.int32(t), msg0, sem_w0)
            else:
                wait_scatter(jnp.int32(t), msg1, sem_w1)
        plsc.subcore_barrier()

        pltpu.sync_copy(sums_sp.at[pl.ds(rbase, _RS), :],
                        sums_out.at[c, pl.ds(rbase, _RS), :])
        pltpu.sync_copy(cnt_sp.at[pl.ds(rbase, _RS)],
                        cnt_out.at[c, pl.ds(rbase, _RS)])

    return _sc_body


def _sc_gather_scatter(x, score, src3, dst3, z2, z1, nchunk):
    mesh = plsc.VectorSubcoreMesh(core_axis_name="c", subcore_axis_name="s")
    f = pl.kernel(
        _make_sc_body(nchunk),
        out_type=[
            jax.ShapeDtypeStruct((_NC, _NPAD, _D), jnp.float32),
            jax.ShapeDtypeStruct((_NC, _NPAD), jnp.float32),
        ],
        mesh=mesh,
        scratch_types=[
            pltpu.VMEM((8, _C), jnp.int32),
            pltpu.VMEM((8, _C), jnp.int32),
            pltpu.VMEM((_C, _D), jnp.float32),
            pltpu.VMEM((_C, _D), jnp.float32),
            pltpu.VMEM((_C, _D), jnp.float32),
            pltpu.VMEM((_C, _D), jnp.float32),
            pltpu.VMEM((_C, _D), jnp.float32),
            pltpu.VMEM((_C, _D), jnp.float32),
            pltpu.VMEM((48,), jnp.float32),
            pltpu.VMEM_SHARED((_NPAD, _D), jnp.float32),
            pltpu.VMEM_SHARED((_NPAD,), jnp.float32),
            pltpu.SemaphoreType.DMA,
            pltpu.SemaphoreType.DMA,
            pltpu.SemaphoreType.DMA,
            pltpu.SemaphoreType.DMA,
            pltpu.SemaphoreType.DMA,
        ],
    )
    return f(x, score, src3, dst3, z2, z1)


_BN = 1000


def _out_body(*refs):
    sums_refs = refs[:_S]
    cnt_ref, WO_ref, bO_ref, out_ref = refs[_S:]
    s = sums_refs[0][0] + sums_refs[0][1]
    for rf in sums_refs[1:]:
        s = s + rf[0] + rf[1]
    cnt = cnt_ref[:, 0:1]
    for k in range(1, 2 * _S):
        cnt = cnt + cnt_ref[:, k:k + 1]
    wv = s / jnp.maximum(cnt, 1.0)
    out_ref[...] = (jnp.dot(wv, WO_ref[...], preferred_element_type=jnp.float32)
                    + bO_ref[...])


def _mean_project(sums_list, cnt_t, WO, bO):
    return pl.pallas_call(
        _out_body,
        grid=(_N // _BN,),
        in_specs=(
            [pl.BlockSpec((_NC, _BN, _D), lambda i: (0, i, 0))] * _S
            + [
                pl.BlockSpec((_BN, 2 * _S), lambda i: (i, 0)),
                pl.BlockSpec((_D, _D), lambda i: (0, 0)),
                pl.BlockSpec((1, _D), lambda i: (0, 0)),
            ]
        ),
        out_specs=pl.BlockSpec((_BN, _D), lambda i: (i, 0)),
        out_shape=jax.ShapeDtypeStruct((_N, _D), jnp.float32),
    )(*sums_list, cnt_t, WO, bO.reshape(1, _D))


def kernel(x, pe_index, pe_val, W_in, b_in, W1, b1, W2, b2, W_out, b_out, WO, bO):
    hid = W_in.shape[1]
    pe8 = jnp.reshape(pe_val, (_E // 8, 8 * _P))
    W_blk = (jnp.eye(8, dtype=jnp.float32)[:, None, :, None]
             * W_in[None, :, None, :]).reshape(8 * _P, 8 * hid)
    src = pe_index[0]
    dst = pe_index[1]
    z2 = jnp.zeros((_RS, _D), jnp.float32)
    z1 = jnp.zeros((_RS,), jnp.float32)

    es = _E // _S
    nchunk = es // _NW // _C
    nblk = es // _BE
    sums_list, cnt_list = [], []
    for i in range(_S):
        score = _edge_mlp(pe8, W_blk, b_in, W1, b1, W2, b2, W_out, b_out,
                          nblk, i * nblk)
        src3 = src[i * es:(i + 1) * es].reshape(_NW, nchunk, _C)
        dst3 = dst[i * es:(i + 1) * es].reshape(_NW, nchunk, _C)
        sums, cnt = _sc_gather_scatter(x, score, src3, dst3, z2, z1, nchunk)
        sums_list.append(sums)
        cnt_list.append(cnt)

    cnt_t = jnp.transpose(jnp.concatenate([c[:, :_N] for c in cnt_list], 0))
    return _mean_project(sums_list, cnt_t, WO, bO)
